# baseline (speedup 1.0000x reference)
import functools

import jax
import jax.numpy as jnp
from jax import lax
from jax.experimental import pallas as pl
from jax.experimental.pallas import tpu as pltpu
from jax.experimental.pallas import tpu_sc as plsc

B = 16384
D = 128
V = 100
P = 50

NC, NS = 2, 16
L = 16
NW = NC * NS
BPW = B // NW
GRP = BPW // L

BLK = 2048


def _fuse_body(wproj_ref, bprop_ref, bproj_ref, wprop_ref, wf_ref, bias_ref):
    wbot = wproj_ref[D:, :]
    bias_ref[...] = bprop_ref[...] @ wbot + bproj_ref[...]
    wf_ref[...] = wprop_ref[...] @ wbot


def _fuse_weights(w_proj, b_prop, b_proj, w_prop):
    return pl.pallas_call(
        _fuse_body,
        out_shape=(
            jax.ShapeDtypeStruct((P, D), jnp.float32),
            jax.ShapeDtypeStruct((1, D), jnp.float32),
        ),
    )(w_proj, b_prop.reshape(1, D), b_proj.reshape(1, D), w_prop)


@functools.cache
def _make_sc_gather():
    mesh = plsc.VectorSubcoreMesh(core_axis_name="c", subcore_axis_name="s",
                                  num_cores=NC, num_subcores=NS)

    @functools.partial(
        pl.kernel,
        out_type=jax.ShapeDtypeStruct((B * D,), jnp.float32),
        mesh=mesh,
        scratch_types=[
            pltpu.VMEM((V * D,), jnp.float32),
            pltpu.VMEM((BPW,), jnp.int32),
            pltpu.VMEM((BPW * D,), jnp.float32),
        ],
        compiler_params=pltpu.CompilerParams(needs_layout_passes=False),
    )
    def _sc_gather(table_hbm, idx_hbm, out_hbm, table_v, idx_v, rows_v):
        wid = lax.axis_index("s") * NC + lax.axis_index("c")
        base = wid * BPW
        pltpu.sync_copy(table_hbm, table_v)
        pltpu.sync_copy(idx_hbm.at[pl.ds(base, BPW)], idx_v)
        lane = lax.iota(jnp.int32, L)

        def group(g, carry):
            ids16 = idx_v[pl.ds(g * L, L)]
            src0 = ids16 * D
            dst0 = (g * L + lane) * D
            for j in range(D):
                vals = plsc.load_gather(table_v, [src0 + j])
                plsc.store_scatter(rows_v, [dst0 + j], vals)
            return carry

        lax.fori_loop(0, GRP, group, 0)
        pltpu.sync_copy(rows_v, out_hbm.at[pl.ds(base * D, BPW * D)])

    return _sc_gather


def _combine_body(wtop_ref, wf_ref, bias_ref, g_ref, prop_ref, out_ref):
    out_ref[...] = (g_ref[...] @ wtop_ref[...] + prop_ref[...] @ wf_ref[...]
                    + bias_ref[...])


def _combine(w_proj, w_fused, bias, g, prop):
    grid = B // BLK
    return pl.pallas_call(
        _combine_body,
        grid=(grid,),
        in_specs=[
            pl.BlockSpec((D, D), lambda i: (0, 0)),
            pl.BlockSpec((P, D), lambda i: (0, 0)),
            pl.BlockSpec((1, D), lambda i: (0, 0)),
            pl.BlockSpec((BLK, D), lambda i: (i, 0)),
            pl.BlockSpec((BLK, P), lambda i: (i, 0)),
        ],
        out_specs=pl.BlockSpec((BLK, D), lambda i: (i, 0)),
        out_shape=jax.ShapeDtypeStruct((B, D), jnp.float32),
    )(w_proj, w_fused, bias, g, prop)


def kernel(event_type_ids, prop_vectors, event_type_table, W_prop, b_prop,
           W_proj, b_proj):
    ids = event_type_ids.astype(jnp.int32)
    g = _make_sc_gather()(event_type_table.reshape(V * D), ids)
    w_fused, bias = _fuse_weights(W_proj, b_prop, b_proj, W_prop)
    return _combine(W_proj, w_fused, bias, g.reshape(B, D), prop_vectors)


# speedup vs baseline: 2.2251x; 2.2251x over previous
import functools

import jax
import jax.numpy as jnp
from jax import lax
from jax.experimental import pallas as pl
from jax.experimental.pallas import tpu as pltpu
from jax.experimental.pallas import tpu_sc as plsc

B = 16384
D = 128
V = 100
P = 50

NC, NS = 2, 16
NW = NC * NS
BPW = B // NW

BLK = 2048


@functools.cache
def _make_sc_gather():
    mesh = plsc.VectorSubcoreMesh(core_axis_name="c", subcore_axis_name="s",
                                  num_cores=NC, num_subcores=NS)

    @functools.partial(
        pl.kernel,
        out_type=jax.ShapeDtypeStruct((B, D), jnp.float32),
        mesh=mesh,
        scratch_types=[
            pltpu.VMEM((BPW,), jnp.int32),
            pltpu.VMEM((BPW, D), jnp.float32),
            pltpu.SemaphoreType.DMA,
        ],
    )
    def _sc_gather(table_hbm, idx_hbm, out_hbm, idx_v, rows_v, sem):
        wid = lax.axis_index("s") * NC + lax.axis_index("c")
        base = wid * BPW
        pltpu.sync_copy(idx_hbm.at[pl.ds(base, BPW)], idx_v)
        pltpu.async_copy(table_hbm.at[idx_v], rows_v, sem).wait()
        pltpu.sync_copy(rows_v, out_hbm.at[pl.ds(base, BPW)])

    return _sc_gather


def _combine_body(wproj_ref, wprop_ref, bprop_ref, bproj_ref, g_ref, prop_ref,
                  out_ref, wf_ref, bias_ref):
    @pl.when(pl.program_id(0) == 0)
    def _():
        wbot = wproj_ref[D:, :]
        wf_ref[...] = wprop_ref[...] @ wbot
        bias_ref[...] = bprop_ref[...] @ wbot + bproj_ref[...]

    out_ref[...] = (g_ref[...] @ wproj_ref[:D, :] + prop_ref[...] @ wf_ref[...]
                    + bias_ref[...])


def _combine(w_proj, w_prop, b_prop, b_proj, g, prop):
    grid = B // BLK
    return pl.pallas_call(
        _combine_body,
        grid=(grid,),
        in_specs=[
            pl.BlockSpec((2 * D, D), lambda i: (0, 0)),
            pl.BlockSpec((P, D), lambda i: (0, 0)),
            pl.BlockSpec((1, D), lambda i: (0, 0)),
            pl.BlockSpec((1, D), lambda i: (0, 0)),
            pl.BlockSpec((BLK, D), lambda i: (i, 0)),
            pl.BlockSpec((BLK, P), lambda i: (i, 0)),
        ],
        out_specs=pl.BlockSpec((BLK, D), lambda i: (i, 0)),
        out_shape=jax.ShapeDtypeStruct((B, D), jnp.float32),
        scratch_shapes=[
            pltpu.VMEM((P, D), jnp.float32),
            pltpu.VMEM((1, D), jnp.float32),
        ],
    )(w_proj, w_prop, b_prop, b_proj, g, prop)


def kernel(event_type_ids, prop_vectors, event_type_table, W_prop, b_prop,
           W_proj, b_proj):
    ids = event_type_ids.astype(jnp.int32)
    g = _make_sc_gather()(event_type_table, ids)
    return _combine(W_proj, W_prop, b_prop.reshape(1, D), b_proj.reshape(1, D),
                    g, prop_vectors)


# speedup vs baseline: 2.3187x; 1.0421x over previous
import functools

import jax
import jax.numpy as jnp
from jax import lax
from jax.experimental import pallas as pl
from jax.experimental.pallas import tpu as pltpu
from jax.experimental.pallas import tpu_sc as plsc

B = 16384
D = 128
V = 100
VPAD = 128
P = 50

NC, NS = 2, 16
NW = NC * NS

B_SC = 8192
BPW = B_SC // NW

BLK = 2048
NB = B // BLK
NSCB = B_SC // BLK


@functools.cache
def _make_sc_gather():
    mesh = plsc.VectorSubcoreMesh(core_axis_name="c", subcore_axis_name="s",
                                  num_cores=NC, num_subcores=NS)

    @functools.partial(
        pl.kernel,
        out_type=jax.ShapeDtypeStruct((B_SC, D), jnp.float32),
        mesh=mesh,
        scratch_types=[
            pltpu.VMEM((BPW,), jnp.int32),
            pltpu.VMEM((BPW, D), jnp.float32),
            pltpu.SemaphoreType.DMA,
        ],
    )
    def _sc_gather(table_hbm, idx_hbm, out_hbm, idx_v, rows_v, sem):
        wid = lax.axis_index("s") * NC + lax.axis_index("c")
        base = wid * BPW
        pltpu.sync_copy(idx_hbm.at[pl.ds(base, BPW)], idx_v)
        pltpu.async_copy(table_hbm.at[idx_v], rows_v, sem).wait()
        pltpu.sync_copy(rows_v, out_hbm.at[pl.ds(base, BPW)])

    return _sc_gather


def _combine_body(wproj_ref, wprop_ref, bprop_ref, bproj_ref, table_ref,
                  ids_ref, g_ref, prop_ref, out_ref, wf_ref, bias_ref):
    i = pl.program_id(0)

    @pl.when(i == 0)
    def _():
        wbot = wproj_ref[D:, :]
        wf_ref[...] = wprop_ref[...] @ wbot
        bias_ref[...] = bprop_ref[...] @ wbot + bproj_ref[...]

    common = prop_ref[...] @ wf_ref[...] + bias_ref[...]
    use_g = i >= NB - NSCB

    @pl.when(use_g)
    def _():
        out_ref[...] = g_ref[...] @ wproj_ref[:D, :] + common

    @pl.when(jnp.logical_not(use_g))
    def _():
        iota = lax.broadcasted_iota(jnp.int32, (1, VPAD), 1)
        onehot = (ids_ref[...] == iota).astype(jnp.float32)
        out_ref[...] = (onehot[:, :V] @ table_ref[...]) @ wproj_ref[:D, :] \
            + common


def _combine(w_proj, w_prop, b_prop, b_proj, table, ids2d, g, prop):
    def bmap(i):
        return (i + NSCB) % NB

    return pl.pallas_call(
        _combine_body,
        grid=(NB,),
        in_specs=[
            pl.BlockSpec((2 * D, D), lambda i: (0, 0)),
            pl.BlockSpec((P, D), lambda i: (0, 0)),
            pl.BlockSpec((1, D), lambda i: (0, 0)),
            pl.BlockSpec((1, D), lambda i: (0, 0)),
            pl.BlockSpec((V, D), lambda i: (0, 0)),
            pl.BlockSpec((BLK, 1), lambda i: (bmap(i), 0)),
            pl.BlockSpec((BLK, D),
                         lambda i: (jnp.maximum(i - (NB - NSCB), 0), 0)),
            pl.BlockSpec((BLK, P), lambda i: (bmap(i), 0)),
        ],
        out_specs=pl.BlockSpec((BLK, D), lambda i: (bmap(i), 0)),
        out_shape=jax.ShapeDtypeStruct((B, D), jnp.float32),
        scratch_shapes=[
            pltpu.VMEM((P, D), jnp.float32),
            pltpu.VMEM((1, D), jnp.float32),
        ],
    )(w_proj, w_prop, b_prop, b_proj, table, ids2d, g, prop)


def kernel(event_type_ids, prop_vectors, event_type_table, W_prop, b_prop,
           W_proj, b_proj):
    ids = event_type_ids.astype(jnp.int32)
    g = _make_sc_gather()(event_type_table, ids)
    return _combine(W_proj, W_prop, b_prop.reshape(1, D), b_proj.reshape(1, D),
                    event_type_table, ids.reshape(B, 1), g, prop_vectors)


# speedup vs baseline: 2.7685x; 1.1940x over previous
import functools

import jax
import jax.numpy as jnp
from jax import lax
from jax.experimental import pallas as pl
from jax.experimental.pallas import tpu as pltpu
from jax.experimental.pallas import tpu_sc as plsc

B = 16384
D = 128
V = 100
VPAD = 128
P = 50

NC, NS = 2, 16
NW = NC * NS

B_SC = 8192
BPW = B_SC // NW

BLK = 2048
NB = B // BLK
NSCB = B_SC // BLK


@functools.cache
def _make_sc_gather():
    mesh = plsc.VectorSubcoreMesh(core_axis_name="c", subcore_axis_name="s",
                                  num_cores=NC, num_subcores=NS)

    @functools.partial(
        pl.kernel,
        out_type=jax.ShapeDtypeStruct((B_SC, D), jnp.float32),
        mesh=mesh,
        scratch_types=[
            pltpu.VMEM((BPW,), jnp.int32),
            pltpu.VMEM((BPW, D), jnp.float32),
            pltpu.SemaphoreType.DMA,
        ],
    )
    def _sc_gather(table_hbm, idx_hbm, out_hbm, idx_v, rows_v, sem):
        wid = lax.axis_index("s") * NC + lax.axis_index("c")
        base = wid * BPW
        pltpu.sync_copy(idx_hbm.at[pl.ds(base, BPW)], idx_v)
        pltpu.async_copy(table_hbm.at[idx_v], rows_v, sem).wait()
        pltpu.sync_copy(rows_v, out_hbm.at[pl.ds(base, BPW)])

    return _sc_gather


def _fuse_into_scratch(i, wproj_ref, wprop_ref, bprop_ref, bproj_ref, wf_ref,
                       bias_ref):
    @pl.when(i == 0)
    def _():
        wbot = wproj_ref[D:, :]
        wf_ref[...] = wprop_ref[...] @ wbot
        bias_ref[...] = bprop_ref[...] @ wbot + bproj_ref[...]


def _onehot_body(wproj_ref, wprop_ref, bprop_ref, bproj_ref, table_ref,
                 ids_ref, prop_ref, out_ref, wf_ref, bias_ref):
    i = pl.program_id(0)
    _fuse_into_scratch(i, wproj_ref, wprop_ref, bprop_ref, bproj_ref, wf_ref,
                       bias_ref)
    common = prop_ref[...] @ wf_ref[...] + bias_ref[...]
    iota_v = lax.broadcasted_iota(jnp.int32, (V, 1), 0)
    oh_t = (ids_ref[0] == iota_v).astype(jnp.float32)
    g_blk = lax.dot_general(oh_t, table_ref[...],
                            (((0,), (0,)), ((), ())))
    out_ref[...] = g_blk @ wproj_ref[:D, :] + common


def _gathered_body(prev_ref, wproj_ref, wprop_ref, bprop_ref, bproj_ref,
                   g_ref, prop_ref, out_ref, wf_ref, bias_ref):
    del prev_ref
    i = pl.program_id(0)
    _fuse_into_scratch(i, wproj_ref, wprop_ref, bprop_ref, bproj_ref, wf_ref,
                       bias_ref)
    common = prop_ref[...] @ wf_ref[...] + bias_ref[...]
    out_ref[...] = g_ref[...] @ wproj_ref[:D, :] + common


_WSPECS = [
    pl.BlockSpec((2 * D, D), lambda i: (0, 0)),
    pl.BlockSpec((P, D), lambda i: (0, 0)),
    pl.BlockSpec((1, D), lambda i: (0, 0)),
    pl.BlockSpec((1, D), lambda i: (0, 0)),
]
_SCRATCH = [
    pltpu.VMEM((P, D), jnp.float32),
    pltpu.VMEM((1, D), jnp.float32),
]
_OUT_SHAPE = jax.ShapeDtypeStruct((B, D), jnp.float32)


def _combine_onehot(w_proj, w_prop, b_prop, b_proj, table, ids3, prop):
    return pl.pallas_call(
        _onehot_body,
        grid=(NB - NSCB,),
        in_specs=_WSPECS + [
            pl.BlockSpec((V, D), lambda i: (0, 0)),
            pl.BlockSpec((1, 1, BLK), lambda i: (i + NSCB, 0, 0)),
            pl.BlockSpec((BLK, P), lambda i: (i + NSCB, 0)),
        ],
        out_specs=pl.BlockSpec((BLK, D), lambda i: (i + NSCB, 0)),
        out_shape=_OUT_SHAPE,
        scratch_shapes=_SCRATCH,
    )(w_proj, w_prop, b_prop, b_proj, table, ids3, prop)


def _combine_gathered(prev, w_proj, w_prop, b_prop, b_proj, g, prop):
    return pl.pallas_call(
        _gathered_body,
        grid=(NSCB,),
        in_specs=[pl.BlockSpec(memory_space=pl.ANY)] + _WSPECS + [
            pl.BlockSpec((BLK, D), lambda i: (i, 0)),
            pl.BlockSpec((BLK, P), lambda i: (i, 0)),
        ],
        out_specs=pl.BlockSpec((BLK, D), lambda i: (i, 0)),
        out_shape=_OUT_SHAPE,
        input_output_aliases={0: 0},
        scratch_shapes=_SCRATCH,
    )(prev, w_proj, w_prop, b_prop, b_proj, g, prop)


def kernel(event_type_ids, prop_vectors, event_type_table, W_prop, b_prop,
           W_proj, b_proj):
    ids = event_type_ids.astype(jnp.int32)
    g = _make_sc_gather()(event_type_table, ids)
    bprop2 = b_prop.reshape(1, D)
    bproj2 = b_proj.reshape(1, D)
    ids3 = ids.reshape(NB, 1, BLK)
    out = _combine_onehot(W_proj, W_prop, bprop2, bproj2, event_type_table,
                          ids3, prop_vectors)
    return _combine_gathered(out, W_proj, W_prop, bprop2, bproj2, g,
                             prop_vectors)
